# baseline (speedup 1.0000x reference)
import functools

import jax
import jax.numpy as jnp
from jax import lax
from jax.experimental import pallas as pl
from jax.experimental.pallas import tpu as pltpu
from jax.experimental.pallas import tpu_sc as plsc

_NUM_CORES = 2
_NUM_SUBCORES = 16
_NW = _NUM_CORES * _NUM_SUBCORES
_LANES = 16
_CH = 512


def _pack_mm_relu(x_cn, w):
    c, n = x_cn.shape

    def body(w_ref, x_ref, o_ref):
        t = jnp.maximum(
            jnp.dot(w_ref[...], x_ref[...],
                    preferred_element_type=jnp.float32),
            0.0)
        tb = t.astype(jnp.bfloat16)
        lo = lax.bitcast_convert_type(tb[:c // 2], jnp.uint16).astype(jnp.uint32)
        hi = lax.bitcast_convert_type(tb[c // 2:], jnp.uint16).astype(jnp.uint32)
        o_ref[...] = lax.bitcast_convert_type(lo | (hi << 16), jnp.int32)

    return pl.pallas_call(
        body,
        in_specs=[
            pl.BlockSpec((c, c), lambda: (0, 0)),
            pl.BlockSpec((c, n), lambda: (0, 0)),
        ],
        out_specs=pl.BlockSpec((c // 2, n), lambda: (0, 0)),
        out_shape=jax.ShapeDtypeStruct((c // 2, n), jnp.int32),
    )(w, x_cn)


def _final(x_cn, aggr_words, wx, wa, bias_col):
    c, n = x_cn.shape
    n_pad = aggr_words.shape[1]

    def body(x_ref, a_ref, wx_ref, wa_ref, b_ref, o_ref):
        words = a_ref[:, :n]
        lo = lax.bitcast_convert_type(words << 16, jnp.float32)
        hi = lax.bitcast_convert_type(words & jnp.int32(-65536), jnp.float32)
        aggr = jnp.concatenate((lo, hi), axis=0)
        t = jnp.dot(wx_ref[...], x_ref[...],
                    preferred_element_type=jnp.float32)
        t += jnp.dot(wa_ref[...], aggr, preferred_element_type=jnp.float32)
        t = jnp.maximum(t, 0.0) + b_ref[...]
        norm = jnp.sqrt(jnp.sum(t * t, axis=0, keepdims=True))
        o_ref[0] = t / jnp.maximum(norm, 1e-12)

    return pl.pallas_call(
        body,
        in_specs=[
            pl.BlockSpec((c, n), lambda: (0, 0)),
            pl.BlockSpec((c // 2, n_pad), lambda: (0, 0)),
            pl.BlockSpec((c, c), lambda: (0, 0)),
            pl.BlockSpec((c, c), lambda: (0, 0)),
            pl.BlockSpec((c, 1), lambda: (0, 0)),
        ],
        out_specs=pl.BlockSpec((1, c, n), lambda: (0, 0, 0)),
        out_shape=jax.ShapeDtypeStruct((1, c, n), jnp.float32),
    )(x_cn, aggr_words, wx, wa, bias_col)


def _sc_gather_max(ztab_flat, idx_flat, n_tab, n_pad, k):
    npk = ztab_flat.size // n_tab
    half = n_pad // 2
    nchunks = half // _CH
    ngroups = _CH // _LANES
    assert half % _CH == 0 and nchunks % 2 == 0 and ngroups % 2 == 0
    mesh = plsc.VectorSubcoreMesh(
        core_axis_name="c", subcore_axis_name="s")

    @functools.partial(
        pl.kernel,
        out_type=jax.ShapeDtypeStruct((npk, n_pad), jnp.int32),
        mesh=mesh,
        compiler_params=pltpu.CompilerParams(
            use_tc_tiling_on_sc=False, needs_layout_passes=False),
        scratch_types=[
            pltpu.VMEM((4 * n_tab,), jnp.int32),
            pltpu.VMEM((2, _CH * k), jnp.int32),
            pltpu.VMEM((2, 4, _CH), jnp.int32),
            [pltpu.SemaphoreType.DMA] * 2,
            [pltpu.SemaphoreType.DMA] * 2,
        ],
    )
    def sc_kernel(z_hbm, idx_hbm, out_hbm, tab_v, idx_v, outb_v, isems, osems):
        wid = lax.axis_index("s") * _NUM_CORES + lax.axis_index("c")
        a0 = (wid % 16) * 4
        nbase = (wid // 16) * half

        pltpu.sync_copy(z_hbm.at[pl.ds(a0 * n_tab, 4 * n_tab)], tab_v)
        tabs = [tab_v.at[pl.ds(a * n_tab, n_tab)] for a in range(4)]
        iota_k = lax.iota(jnp.int32, _LANES) * k

        def idma(ci, s):
            start = jnp.minimum(nbase + ci * _CH, n_tab - _CH)
            return pltpu.make_async_copy(
                idx_hbm.at[pl.ds(start * k, _CH * k)],
                idx_v.at[s], isems[s])

        def odma(ci, s):
            return pltpu.make_async_copy(
                outb_v.at[s],
                out_hbm.at[pl.ds(a0, 4), pl.ds(nbase + ci * _CH, _CH)],
                osems[s])

        idma(0, 0).start()
        idma(1, 1).start()

        def chunk_pair(g, carry):
            for s in range(2):
                ci = g * 2 + s
                idma(ci, s).wait()

                @pl.when(ci >= 2)
                def _():
                    odma(ci - 2, s).wait()

                ib = idx_v.at[s]
                start = nbase + ci * _CH
                off_k = (start - jnp.minimum(start, n_tab - _CH)) * k

                @functools.partial(plsc.parallel_loop, 0, ngroups, unroll=2)
                def _(gp):
                    base = gp * (_LANES * k) + off_k
                    nids = [plsc.load_gather(ib, [iota_k + (base + kk)])
                            for kk in range(k)]
                    sl = pl.ds(gp * _LANES, _LANES)
                    for a in range(4):
                        acc = plsc.bitcast(
                            plsc.load_gather(tabs[a], [nids[0]]),
                            jnp.bfloat16)
                        for kk in range(1, k):
                            v = plsc.bitcast(
                                plsc.load_gather(tabs[a], [nids[kk]]),
                                jnp.bfloat16)
                            acc = jnp.maximum(acc, v)
                        outb_v[s, a, sl] = plsc.bitcast(acc, jnp.int32)

                odma(ci, s).start()

                @pl.when(ci + 2 < nchunks)
                def _():
                    idma(ci + 2, s).start()
            return carry

        lax.fori_loop(0, nchunks // 2, chunk_pair, 0)
        odma(nchunks - 2, 0).wait()
        odma(nchunks - 1, 1).wait()

    return sc_kernel(ztab_flat, idx_flat)


def kernel(x, x_0, edge_index, W_pre, W_nn, bias):
    del x_0
    b, c, n, _ = x.shape
    k = edge_index.shape[-1]
    n_pad = ((n + 2 * _CH - 1) // (2 * _CH)) * (2 * _CH)

    idx_flat = edge_index.reshape(-1)

    x_cn = x[0, :, :, 0]
    ztab = _pack_mm_relu(x_cn, W_pre)
    aggr_words = _sc_gather_max(ztab.reshape(-1), idx_flat, n, n_pad, k)
    out = _final(x_cn, aggr_words, W_nn[:, :c], W_nn[:, c:],
                 bias.reshape(c, 1))
    return out.reshape(b, c, n, 1)


# speedup vs baseline: 1.0513x; 1.0513x over previous
import functools

import jax
import jax.numpy as jnp
from jax import lax
from jax.experimental import pallas as pl
from jax.experimental.pallas import tpu as pltpu
from jax.experimental.pallas import tpu_sc as plsc

_NUM_CORES = 2
_NUM_SUBCORES = 16
_NW = _NUM_CORES * _NUM_SUBCORES
_LANES = 16
_CH = 512


def _pack_mm_relu(x_cn, w):
    c, n = x_cn.shape

    def body(w_ref, x_ref, o_ref):
        t = jnp.maximum(
            jnp.dot(w_ref[...], x_ref[...],
                    preferred_element_type=jnp.float32),
            0.0)
        tb = t.astype(jnp.bfloat16)
        lo = lax.bitcast_convert_type(tb[:c // 2], jnp.uint16).astype(jnp.uint32)
        hi = lax.bitcast_convert_type(tb[c // 2:], jnp.uint16).astype(jnp.uint32)
        o_ref[...] = lax.bitcast_convert_type(lo | (hi << 16), jnp.int32)

    return pl.pallas_call(
        body,
        in_specs=[
            pl.BlockSpec((c, c), lambda: (0, 0)),
            pl.BlockSpec((c, n), lambda: (0, 0)),
        ],
        out_specs=pl.BlockSpec((c // 2, n), lambda: (0, 0)),
        out_shape=jax.ShapeDtypeStruct((c // 2, n), jnp.int32),
    )(w, x_cn)


def _final(x_cn, aggr_words, wx, wa, bias_col):
    c, n = x_cn.shape
    n_pad = aggr_words.shape[1]

    def body(x_ref, a_ref, wx_ref, wa_ref, b_ref, o_ref):
        words = a_ref[:, :n]
        lo = lax.bitcast_convert_type(words << 16, jnp.float32)
        hi = lax.bitcast_convert_type(words & jnp.int32(-65536), jnp.float32)
        aggr = jnp.concatenate((lo, hi), axis=0)
        t = jnp.dot(wx_ref[...], x_ref[...],
                    preferred_element_type=jnp.float32)
        t += jnp.dot(wa_ref[...], aggr, preferred_element_type=jnp.float32)
        t = jnp.maximum(t, 0.0) + b_ref[...]
        norm = jnp.sqrt(jnp.sum(t * t, axis=0, keepdims=True))
        o_ref[0] = t / jnp.maximum(norm, 1e-12)

    return pl.pallas_call(
        body,
        in_specs=[
            pl.BlockSpec((c, n), lambda: (0, 0)),
            pl.BlockSpec((c // 2, n_pad), lambda: (0, 0)),
            pl.BlockSpec((c, c), lambda: (0, 0)),
            pl.BlockSpec((c, c), lambda: (0, 0)),
            pl.BlockSpec((c, 1), lambda: (0, 0)),
        ],
        out_specs=pl.BlockSpec((1, c, n), lambda: (0, 0, 0)),
        out_shape=jax.ShapeDtypeStruct((1, c, n), jnp.float32),
    )(x_cn, aggr_words, wx, wa, bias_col)


def _sc_gather_max(ztab_flat, idx_flat, n_tab, n_pad, k):
    npk = ztab_flat.size // n_tab
    half = n_pad // 2
    nchunks = half // _CH
    ngroups = _CH // _LANES
    assert half % _CH == 0 and nchunks % 2 == 0 and ngroups % 2 == 0
    mesh = plsc.VectorSubcoreMesh(
        core_axis_name="c", subcore_axis_name="s")

    @functools.partial(
        pl.kernel,
        out_type=jax.ShapeDtypeStruct((npk, n_pad), jnp.int32),
        mesh=mesh,
        compiler_params=pltpu.CompilerParams(
            use_tc_tiling_on_sc=False, needs_layout_passes=False),
        scratch_types=[
            pltpu.VMEM((4 * n_tab,), jnp.int32),
            pltpu.VMEM((2, _CH * k), jnp.int32),
            pltpu.VMEM((2, 4, _CH), jnp.int32),
            [pltpu.SemaphoreType.DMA] * 2,
            [pltpu.SemaphoreType.DMA] * 2,
        ],
    )
    def sc_kernel(z_hbm, idx_hbm, out_hbm, tab_v, idx_v, outb_v, isems, osems):
        wid = lax.axis_index("s") * _NUM_CORES + lax.axis_index("c")
        a0 = (wid % 16) * 4
        nbase = (wid // 16) * half

        pltpu.sync_copy(z_hbm.at[pl.ds(a0 * n_tab, 4 * n_tab)], tab_v)
        tabs = [tab_v.at[pl.ds(a * n_tab, n_tab)] for a in range(4)]
        iota_k = lax.iota(jnp.int32, _LANES) * k

        def idma(ci, s):
            start = jnp.minimum(nbase + ci * _CH, n_tab - _CH)
            return pltpu.make_async_copy(
                idx_hbm.at[pl.ds(start * k, _CH * k)],
                idx_v.at[s], isems[s])

        def odma(ci, s):
            return pltpu.make_async_copy(
                outb_v.at[s],
                out_hbm.at[pl.ds(a0, 4), pl.ds(nbase + ci * _CH, _CH)],
                osems[s])

        idma(0, 0).start()
        idma(1, 1).start()

        def chunk_pair(g, carry):
            for s in range(2):
                ci = g * 2 + s
                idma(ci, s).wait()

                @pl.when(ci >= 2)
                def _():
                    odma(ci - 2, s).wait()

                ib = idx_v.at[s]
                start = nbase + ci * _CH
                off_k = (start - jnp.minimum(start, n_tab - _CH)) * k

                @functools.partial(plsc.parallel_loop, 0, ngroups, unroll=2)
                def _(gp):
                    base = gp * (_LANES * k) + off_k
                    nids = [plsc.load_gather(ib, [iota_k + (base + kk)])
                            for kk in range(k)]
                    sl = pl.ds(gp * _LANES, _LANES)
                    for a in range(4):
                        acc = plsc.bitcast(
                            plsc.load_gather(tabs[a], [nids[0]]),
                            jnp.bfloat16)
                        for kk in range(1, k):
                            v = plsc.bitcast(
                                plsc.load_gather(tabs[a], [nids[kk]]),
                                jnp.bfloat16)
                            acc = jnp.maximum(acc, v)
                        outb_v[s, a, sl] = plsc.bitcast(acc, jnp.int32)

                odma(ci, s).start()

                @pl.when(ci + 2 < nchunks)
                def _():
                    idma(ci + 2, s).start()
            return carry

        lax.fori_loop(0, nchunks // 2, chunk_pair, 0)
        odma(nchunks - 2, 0).wait()
        odma(nchunks - 1, 1).wait()

    return sc_kernel(ztab_flat, idx_flat)


def kernel(x, x_0, edge_index, W_pre, W_nn, bias):
    del x_0
    b, c, n, _ = x.shape
    k = edge_index.shape[-1]
    n_pad = ((n + 2 * _CH - 1) // (2 * _CH)) * (2 * _CH)

    idx_flat = edge_index[0, 0].reshape(-1)

    x_cn = x[0, :, :, 0]
    ztab = _pack_mm_relu(x_cn, W_pre)
    aggr_words = _sc_gather_max(ztab.reshape(-1), idx_flat, n, n_pad, k)
    out = _final(x_cn, aggr_words, W_nn[:, :c], W_nn[:, c:],
                 bias.reshape(c, 1))
    return out.reshape(b, c, n, 1)


# speedup vs baseline: 1.0769x; 1.0243x over previous
import functools

import jax
import jax.numpy as jnp
from jax import lax
from jax.experimental import pallas as pl
from jax.experimental.pallas import tpu as pltpu
from jax.experimental.pallas import tpu_sc as plsc

_NUM_CORES = 2
_NUM_SUBCORES = 16
_NW = _NUM_CORES * _NUM_SUBCORES
_LANES = 16
_CH = 512


def _pack_mm_relu(x_cn, w):
    c, n = x_cn.shape

    def body(w_ref, x_ref, o_ref):
        t = jnp.maximum(
            jnp.dot(w_ref[...], x_ref[...],
                    preferred_element_type=jnp.float32),
            0.0)
        tb = t.astype(jnp.bfloat16)
        lo = lax.bitcast_convert_type(tb[:c // 2], jnp.uint16).astype(jnp.uint32)
        hi = lax.bitcast_convert_type(tb[c // 2:], jnp.uint16).astype(jnp.uint32)
        words = lax.bitcast_convert_type(lo | (hi << 16), jnp.int32)
        for r in range(c // 2):
            o_ref[pl.ds(r * n, n)] = words[r]

    return pl.pallas_call(
        body,
        in_specs=[
            pl.BlockSpec((c, c), lambda: (0, 0)),
            pl.BlockSpec((c, n), lambda: (0, 0)),
        ],
        out_specs=pl.BlockSpec((c // 2 * n,), lambda: (0,)),
        out_shape=jax.ShapeDtypeStruct((c // 2 * n,), jnp.int32),
    )(w, x_cn)


def _final(x_cn, aggr_words, wx, wa, bias_col):
    c, n = x_cn.shape
    n_pad = aggr_words.shape[1]

    def body(x_ref, a_ref, wx_ref, wa_ref, b_ref, o_ref):
        words = a_ref[:, :n]
        lo = lax.bitcast_convert_type(words << 16, jnp.float32)
        hi = lax.bitcast_convert_type(words & jnp.int32(-65536), jnp.float32)
        aggr = jnp.concatenate((lo, hi), axis=0)
        t = jnp.dot(wx_ref[...], x_ref[...],
                    preferred_element_type=jnp.float32)
        t += jnp.dot(wa_ref[...], aggr, preferred_element_type=jnp.float32)
        t = jnp.maximum(t, 0.0) + b_ref[...]
        norm = jnp.sqrt(jnp.sum(t * t, axis=0, keepdims=True))
        res = t / jnp.maximum(norm, 1e-12)
        for r in range(c):
            o_ref[pl.ds(r * n, n)] = res[r]

    return pl.pallas_call(
        body,
        in_specs=[
            pl.BlockSpec((c, n), lambda: (0, 0)),
            pl.BlockSpec((c // 2, n_pad), lambda: (0, 0)),
            pl.BlockSpec((c, c), lambda: (0, 0)),
            pl.BlockSpec((c, c), lambda: (0, 0)),
            pl.BlockSpec((c, 1), lambda: (0, 0)),
        ],
        out_specs=pl.BlockSpec((c * n,), lambda: (0,)),
        out_shape=jax.ShapeDtypeStruct((c * n,), jnp.float32),
    )(x_cn, aggr_words, wx, wa, bias_col)


def _sc_gather_max(ztab_flat, idx_flat, n_tab, n_pad, k):
    npk = ztab_flat.size // n_tab
    half = n_pad // 2
    nchunks = half // _CH
    ngroups = _CH // _LANES
    assert half % _CH == 0 and nchunks % 2 == 0 and ngroups % 2 == 0
    mesh = plsc.VectorSubcoreMesh(
        core_axis_name="c", subcore_axis_name="s")

    @functools.partial(
        pl.kernel,
        out_type=jax.ShapeDtypeStruct((npk, n_pad), jnp.int32),
        mesh=mesh,
        compiler_params=pltpu.CompilerParams(
            use_tc_tiling_on_sc=False, needs_layout_passes=False),
        scratch_types=[
            pltpu.VMEM((4 * n_tab,), jnp.int32),
            pltpu.VMEM((2, _CH * k), jnp.int32),
            pltpu.VMEM((2, 4, _CH), jnp.int32),
            [pltpu.SemaphoreType.DMA] * 2,
            [pltpu.SemaphoreType.DMA] * 2,
        ],
    )
    def sc_kernel(z_hbm, idx_hbm, out_hbm, tab_v, idx_v, outb_v, isems, osems):
        wid = lax.axis_index("s") * _NUM_CORES + lax.axis_index("c")
        a0 = (wid % 16) * 4
        nbase = (wid // 16) * half

        pltpu.sync_copy(z_hbm.at[pl.ds(a0 * n_tab, 4 * n_tab)], tab_v)
        tabs = [tab_v.at[pl.ds(a * n_tab, n_tab)] for a in range(4)]
        iota_k = lax.iota(jnp.int32, _LANES) * k

        def idma(ci, s):
            start = jnp.minimum(nbase + ci * _CH, n_tab - _CH)
            return pltpu.make_async_copy(
                idx_hbm.at[pl.ds(start * k, _CH * k)],
                idx_v.at[s], isems[s])

        def odma(ci, s):
            return pltpu.make_async_copy(
                outb_v.at[s],
                out_hbm.at[pl.ds(a0, 4), pl.ds(nbase + ci * _CH, _CH)],
                osems[s])

        idma(0, 0).start()
        idma(1, 1).start()

        def chunk_pair(g, carry):
            for s in range(2):
                ci = g * 2 + s
                idma(ci, s).wait()

                @pl.when(ci >= 2)
                def _():
                    odma(ci - 2, s).wait()

                ib = idx_v.at[s]
                start = nbase + ci * _CH
                off_k = (start - jnp.minimum(start, n_tab - _CH)) * k

                @functools.partial(plsc.parallel_loop, 0, ngroups, unroll=2)
                def _(gp):
                    base = gp * (_LANES * k) + off_k
                    nids = [plsc.load_gather(ib, [iota_k + (base + kk)])
                            for kk in range(k)]
                    sl = pl.ds(gp * _LANES, _LANES)
                    for a in range(4):
                        acc = plsc.bitcast(
                            plsc.load_gather(tabs[a], [nids[0]]),
                            jnp.bfloat16)
                        for kk in range(1, k):
                            v = plsc.bitcast(
                                plsc.load_gather(tabs[a], [nids[kk]]),
                                jnp.bfloat16)
                            acc = jnp.maximum(acc, v)
                        outb_v[s, a, sl] = plsc.bitcast(acc, jnp.int32)

                odma(ci, s).start()

                @pl.when(ci + 2 < nchunks)
                def _():
                    idma(ci + 2, s).start()
            return carry

        lax.fori_loop(0, nchunks // 2, chunk_pair, 0)
        odma(nchunks - 2, 0).wait()
        odma(nchunks - 1, 1).wait()

    return sc_kernel(ztab_flat, idx_flat)


def kernel(x, x_0, edge_index, W_pre, W_nn, bias):
    del x_0
    b, c, n, _ = x.shape
    k = edge_index.shape[-1]
    n_pad = ((n + 2 * _CH - 1) // (2 * _CH)) * (2 * _CH)

    idx_flat = edge_index[0, 0].reshape(-1)

    x_cn = x[0, :, :, 0]
    ztab = _pack_mm_relu(x_cn, W_pre)
    aggr_words = _sc_gather_max(ztab, idx_flat, n, n_pad, k)
    out = _final(x_cn, aggr_words, W_nn[:, :c], W_nn[:, c:],
                 bias.reshape(c, 1))
    return out.reshape(b, c, n, 1)


# speedup vs baseline: 1.1003x; 1.0218x over previous
import functools

import jax
import jax.numpy as jnp
from jax import lax
from jax.experimental import pallas as pl
from jax.experimental.pallas import tpu as pltpu
from jax.experimental.pallas import tpu_sc as plsc

_NUM_CORES = 2
_NUM_SUBCORES = 16
_NW = _NUM_CORES * _NUM_SUBCORES
_LANES = 16
_CH = 512


def _pack_mm_relu(x_cn, w):
    c, n = x_cn.shape

    def body(w_ref, x_ref, o_ref):
        t = jnp.maximum(
            jnp.dot(w_ref[...], x_ref[...],
                    preferred_element_type=jnp.float32),
            0.0)
        tb = t.astype(jnp.bfloat16)
        lo = lax.bitcast_convert_type(tb[:c // 2], jnp.uint16).astype(jnp.uint32)
        hi = lax.bitcast_convert_type(tb[c // 2:], jnp.uint16).astype(jnp.uint32)
        words = lax.bitcast_convert_type(lo | (hi << 16), jnp.int32)
        for r in range(c // 2):
            o_ref[pl.ds(r * n, n)] = words[r]

    return pl.pallas_call(
        body,
        in_specs=[
            pl.BlockSpec((c, c), lambda: (0, 0)),
            pl.BlockSpec((c, n), lambda: (0, 0)),
        ],
        out_specs=pl.BlockSpec((c // 2 * n,), lambda: (0,)),
        out_shape=jax.ShapeDtypeStruct((c // 2 * n,), jnp.int32),
    )(w, x_cn)


def _final(x_cn, aggr_words, wx, wa, bias_col):
    c, n = x_cn.shape
    n_pad = aggr_words.shape[1]

    def body(x_ref, a_ref, wx_ref, wa_ref, b_ref, o_ref):
        words = a_ref[:, :n]
        lo = lax.bitcast_convert_type(words << 16, jnp.float32)
        hi = lax.bitcast_convert_type(words & jnp.int32(-65536), jnp.float32)
        aggr = jnp.concatenate((lo, hi), axis=0)
        t = jnp.dot(wx_ref[...], x_ref[...],
                    preferred_element_type=jnp.float32)
        t += jnp.dot(wa_ref[...], aggr, preferred_element_type=jnp.float32)
        t = jnp.maximum(t, 0.0) + b_ref[...]
        norm = jnp.sqrt(jnp.sum(t * t, axis=0, keepdims=True))
        o_ref[0] = t / jnp.maximum(norm, 1e-12)

    return pl.pallas_call(
        body,
        in_specs=[
            pl.BlockSpec((c, n), lambda: (0, 0)),
            pl.BlockSpec((c // 2, n_pad), lambda: (0, 0)),
            pl.BlockSpec((c, c), lambda: (0, 0)),
            pl.BlockSpec((c, c), lambda: (0, 0)),
            pl.BlockSpec((c, 1), lambda: (0, 0)),
        ],
        out_specs=pl.BlockSpec((1, c, n), lambda: (0, 0, 0)),
        out_shape=jax.ShapeDtypeStruct((1, c, n), jnp.float32),
    )(x_cn, aggr_words, wx, wa, bias_col)


def _sc_gather_max(ztab_flat, idx_flat, n_tab, n_pad, k):
    npk = ztab_flat.size // n_tab
    half = n_pad // 2
    nchunks = half // _CH
    ngroups = _CH // _LANES
    assert half % _CH == 0 and nchunks % 2 == 0 and ngroups % 2 == 0
    mesh = plsc.VectorSubcoreMesh(
        core_axis_name="c", subcore_axis_name="s")

    @functools.partial(
        pl.kernel,
        out_type=jax.ShapeDtypeStruct((npk, n_pad), jnp.int32),
        mesh=mesh,
        compiler_params=pltpu.CompilerParams(
            use_tc_tiling_on_sc=False, needs_layout_passes=False),
        scratch_types=[
            pltpu.VMEM((4 * n_tab,), jnp.int32),
            pltpu.VMEM((2, _CH * k), jnp.int32),
            pltpu.VMEM((2, 4, _CH), jnp.int32),
            [pltpu.SemaphoreType.DMA] * 2,
            [pltpu.SemaphoreType.DMA] * 2,
        ],
    )
    def sc_kernel(z_hbm, idx_hbm, out_hbm, tab_v, idx_v, outb_v, isems, osems):
        wid = lax.axis_index("s") * _NUM_CORES + lax.axis_index("c")
        a0 = (wid % 16) * 4
        nbase = (wid // 16) * half

        pltpu.sync_copy(z_hbm.at[pl.ds(a0 * n_tab, 4 * n_tab)], tab_v)
        tabs = [tab_v.at[pl.ds(a * n_tab, n_tab)] for a in range(4)]
        iota_k = lax.iota(jnp.int32, _LANES) * k

        def idma(ci, s):
            start = jnp.minimum(nbase + ci * _CH, n_tab - _CH)
            return pltpu.make_async_copy(
                idx_hbm.at[pl.ds(start * k, _CH * k)],
                idx_v.at[s], isems[s])

        def odma(ci, s):
            return pltpu.make_async_copy(
                outb_v.at[s],
                out_hbm.at[pl.ds(a0, 4), pl.ds(nbase + ci * _CH, _CH)],
                osems[s])

        idma(0, 0).start()
        idma(1, 1).start()

        def chunk_pair(g, carry):
            for s in range(2):
                ci = g * 2 + s
                idma(ci, s).wait()

                @pl.when(ci >= 2)
                def _():
                    odma(ci - 2, s).wait()

                ib = idx_v.at[s]
                start = nbase + ci * _CH
                off_k = (start - jnp.minimum(start, n_tab - _CH)) * k

                @functools.partial(plsc.parallel_loop, 0, ngroups, unroll=2)
                def _(gp):
                    base = gp * (_LANES * k) + off_k
                    nids = [plsc.load_gather(ib, [iota_k + (base + kk)])
                            for kk in range(k)]
                    sl = pl.ds(gp * _LANES, _LANES)
                    for a in range(4):
                        acc = plsc.bitcast(
                            plsc.load_gather(tabs[a], [nids[0]]),
                            jnp.bfloat16)
                        for kk in range(1, k):
                            v = plsc.bitcast(
                                plsc.load_gather(tabs[a], [nids[kk]]),
                                jnp.bfloat16)
                            acc = jnp.maximum(acc, v)
                        outb_v[s, a, sl] = plsc.bitcast(acc, jnp.int32)

                odma(ci, s).start()

                @pl.when(ci + 2 < nchunks)
                def _():
                    idma(ci + 2, s).start()
            return carry

        lax.fori_loop(0, nchunks // 2, chunk_pair, 0)
        odma(nchunks - 2, 0).wait()
        odma(nchunks - 1, 1).wait()

    return sc_kernel(ztab_flat, idx_flat)


def kernel(x, x_0, edge_index, W_pre, W_nn, bias):
    del x_0
    b, c, n, _ = x.shape
    k = edge_index.shape[-1]
    n_pad = ((n + 2 * _CH - 1) // (2 * _CH)) * (2 * _CH)

    x_cn = x[0, :, :, 0]
    idx_flat = edge_index[0, 0].reshape(-1)
    ztab = _pack_mm_relu(x_cn, W_pre)
    aggr_words = _sc_gather_max(ztab, idx_flat, n, n_pad, k)
    out = _final(x_cn, aggr_words, W_nn[:, :c], W_nn[:, c:],
                 bias.reshape(c, 1))
    return out.reshape(b, c, n, 1)


# speedup vs baseline: 1.1029x; 1.0024x over previous
import functools

import jax
import jax.numpy as jnp
from jax import lax
from jax.experimental import pallas as pl
from jax.experimental.pallas import tpu as pltpu
from jax.experimental.pallas import tpu_sc as plsc

_NUM_CORES = 2
_NUM_SUBCORES = 16
_NW = _NUM_CORES * _NUM_SUBCORES
_LANES = 16
_CH = 512


def _pack_mm_relu(x_cn, w):
    c, n = x_cn.shape

    def body(w_ref, x_ref, o_ref):
        t = jnp.maximum(
            jnp.dot(w_ref[...], x_ref[...],
                    preferred_element_type=jnp.float32),
            0.0)
        tb = t.astype(jnp.bfloat16)
        lo = lax.bitcast_convert_type(tb[:c // 2], jnp.uint16).astype(jnp.uint32)
        hi = lax.bitcast_convert_type(tb[c // 2:], jnp.uint16).astype(jnp.uint32)
        words = lax.bitcast_convert_type(lo | (hi << 16), jnp.int32)
        for r in range(c // 2):
            o_ref[pl.ds(r * n, n)] = words[r]

    return pl.pallas_call(
        body,
        in_specs=[
            pl.BlockSpec((c, c), lambda: (0, 0)),
            pl.BlockSpec((c, n), lambda: (0, 0)),
        ],
        out_specs=pl.BlockSpec((c // 2 * n,), lambda: (0,)),
        out_shape=jax.ShapeDtypeStruct((c // 2 * n,), jnp.int32),
    )(w, x_cn)


def _final(x_cn, aggr_words, wx, wa, bias_col):
    c, n = x_cn.shape
    n_pad = aggr_words.shape[1]

    def body(x_ref, a_ref, wx_ref, wa_ref, b_ref, o_ref):
        words = a_ref[:, :n]
        lo = lax.bitcast_convert_type(words << 16, jnp.float32)
        hi = lax.bitcast_convert_type(words & jnp.int32(-65536), jnp.float32)
        aggr = jnp.concatenate((lo, hi), axis=0)
        t = jnp.dot(wx_ref[...], x_ref[...],
                    preferred_element_type=jnp.float32)
        t += jnp.dot(wa_ref[...], aggr, preferred_element_type=jnp.float32)
        t = jnp.maximum(t, 0.0) + b_ref[...]
        norm = jnp.sqrt(jnp.sum(t * t, axis=0, keepdims=True))
        o_ref[0] = t / jnp.maximum(norm, 1e-12)

    return pl.pallas_call(
        body,
        in_specs=[
            pl.BlockSpec((c, n), lambda: (0, 0)),
            pl.BlockSpec((c // 2, n_pad), lambda: (0, 0)),
            pl.BlockSpec((c, c), lambda: (0, 0)),
            pl.BlockSpec((c, c), lambda: (0, 0)),
            pl.BlockSpec((c, 1), lambda: (0, 0)),
        ],
        out_specs=pl.BlockSpec((1, c, n), lambda: (0, 0, 0)),
        out_shape=jax.ShapeDtypeStruct((1, c, n), jnp.float32),
    )(x_cn, aggr_words, wx, wa, bias_col)


def _sc_gather_max(ztab_flat, idx_flat, n_tab, n_pad, k):
    npk = ztab_flat.size // n_tab
    half = n_pad // 2
    nchunks = half // _CH
    ngroups = _CH // _LANES
    assert half % _CH == 0 and nchunks % 2 == 0 and ngroups % 2 == 0
    mesh = plsc.VectorSubcoreMesh(
        core_axis_name="c", subcore_axis_name="s")

    @functools.partial(
        pl.kernel,
        out_type=jax.ShapeDtypeStruct((npk, n_pad), jnp.int32),
        mesh=mesh,
        compiler_params=pltpu.CompilerParams(
            use_tc_tiling_on_sc=False, needs_layout_passes=False),
        scratch_types=[
            pltpu.VMEM((4 * n_tab,), jnp.int32),
            pltpu.VMEM((2, _CH * k), jnp.int32),
            pltpu.VMEM((2, 4, _CH), jnp.int32),
            [pltpu.SemaphoreType.DMA] * 2,
            [pltpu.SemaphoreType.DMA] * 2,
        ],
    )
    def sc_kernel(z_hbm, idx_hbm, out_hbm, tab_v, idx_v, outb_v, isems, osems):
        wid = lax.axis_index("s") * _NUM_CORES + lax.axis_index("c")
        a0 = (wid % 16) * 4
        nbase = (wid // 16) * half

        pltpu.sync_copy(z_hbm.at[pl.ds(a0 * n_tab, 4 * n_tab)], tab_v)
        tabs = [tab_v.at[pl.ds(a * n_tab, n_tab)] for a in range(4)]
        iota_k = lax.iota(jnp.int32, _LANES) * k

        def idma(ci, s):
            start = jnp.minimum(nbase + ci * _CH, n_tab - _CH)
            return pltpu.make_async_copy(
                idx_hbm.at[pl.ds(start * k, _CH * k)],
                idx_v.at[s], isems[s])

        def odma(ci, s):
            return pltpu.make_async_copy(
                outb_v.at[s],
                out_hbm.at[pl.ds(a0, 4), pl.ds(nbase + ci * _CH, _CH)],
                osems[s])

        idma(0, 0).start()
        idma(1, 1).start()

        def chunk_pair(g, carry):
            for s in range(2):
                ci = g * 2 + s
                idma(ci, s).wait()

                @pl.when(ci >= 2)
                def _():
                    odma(ci - 2, s).wait()

                ib = idx_v.at[s]
                start = nbase + ci * _CH
                off_k = (start - jnp.minimum(start, n_tab - _CH)) * k

                @functools.partial(plsc.parallel_loop, 0, ngroups, unroll=2)
                def _(gp):
                    base = gp * (_LANES * k) + off_k
                    nids = [jnp.clip(
                        plsc.load_gather(
                            ib, [jnp.minimum(iota_k + (base + kk),
                                             _CH * k - 1)]),
                        0, n_tab - 1) for kk in range(k)]
                    sl = pl.ds(gp * _LANES, _LANES)
                    for a in range(4):
                        acc = plsc.bitcast(
                            plsc.load_gather(tabs[a], [nids[0]]),
                            jnp.bfloat16)
                        for kk in range(1, k):
                            v = plsc.bitcast(
                                plsc.load_gather(tabs[a], [nids[kk]]),
                                jnp.bfloat16)
                            acc = jnp.maximum(acc, v)
                        outb_v[s, a, sl] = plsc.bitcast(acc, jnp.int32)

                odma(ci, s).start()

                @pl.when(ci + 2 < nchunks)
                def _():
                    idma(ci + 2, s).start()
            return carry

        lax.fori_loop(0, nchunks // 2, chunk_pair, 0)
        odma(nchunks - 2, 0).wait()
        odma(nchunks - 1, 1).wait()

    return sc_kernel(ztab_flat, idx_flat)


def kernel(x, x_0, edge_index, W_pre, W_nn, bias):
    del x_0
    b, c, n, _ = x.shape
    k = edge_index.shape[-1]
    n_pad = ((n + 2 * _CH - 1) // (2 * _CH)) * (2 * _CH)

    x_cn = x[0, :, :, 0]
    idx_flat = edge_index[0, 0].reshape(-1)
    ztab = _pack_mm_relu(x_cn, W_pre)
    aggr_words = _sc_gather_max(ztab, idx_flat, n, n_pad, k)
    out = _final(x_cn, aggr_words, W_nn[:, :c], W_nn[:, c:],
                 bias.reshape(c, 1))
    return out.reshape(b, c, n, 1)
